# baseline (speedup 1.0000x reference)
import jax
import jax.numpy as jnp
from jax.experimental import pallas as pl
from jax.experimental.pallas import tpu as pltpu

_R, _C = 200, 16384
_MCW = 2048
_NM = _C // _MCW
_CW = 1024
_NC = _C // _CW
_NS = 4


def _k(s_hbm, m_hbm, o_hbm, m_v, s_b, o_b, sem_m, sem_s, sem_o):
    for q in range(_NM):
        sl = pl.ds(q * _MCW, _MCW)
        pltpu.make_async_copy(m_hbm.at[:, sl], m_v.at[:, sl], sem_m.at[q]).start()
    for c in range(_NS):
        sl = pl.ds(c * _CW, _CW)
        pltpu.make_async_copy(s_hbm.at[:, sl], s_b.at[c], sem_s.at[c]).start()

    cnt = 0.0
    for q in range(_NM):
        sl = pl.ds(q * _MCW, _MCW)
        pltpu.make_async_copy(m_hbm.at[:, sl], m_v.at[:, sl], sem_m.at[q]).wait()
        cnt = cnt + jnp.sum((m_v[:, sl] > 0).astype(jnp.float32))
    scale = 0.6931471805599453 / cnt

    for c in range(_NC):
        cur = c % _NS
        sl = pl.ds(c * _CW, _CW)
        pltpu.make_async_copy(s_hbm.at[:, sl], s_b.at[cur], sem_s.at[cur]).wait()
        if c >= 2:
            psl = pl.ds((c - 2) * _CW, _CW)
            pltpu.make_async_copy(o_b.at[c % 2], o_hbm.at[:, psl], sem_o.at[c % 2]).wait()
        t = jnp.exp2(s_b[cur] * (-1.4426950408889634))
        o_b[c % 2] = (jnp.log2(1.0 + t) * m_v[:, sl]) * scale
        pltpu.make_async_copy(o_b.at[c % 2], o_hbm.at[:, sl], sem_o.at[c % 2]).start()
        if c + _NS < _NC:
            nsl = pl.ds((c + _NS) * _CW, _CW)
            pltpu.make_async_copy(s_hbm.at[:, nsl], s_b.at[cur], sem_s.at[cur]).start()

    for c in (_NC - 2, _NC - 1):
        sl = pl.ds(c * _CW, _CW)
        pltpu.make_async_copy(o_b.at[c % 2], o_hbm.at[:, sl], sem_o.at[c % 2]).wait()


def kernel(output_scores, mask):
    out_t = pl.pallas_call(
        _k,
        in_specs=[
            pl.BlockSpec(memory_space=pltpu.HBM),
            pl.BlockSpec(memory_space=pltpu.HBM),
        ],
        out_specs=pl.BlockSpec(memory_space=pltpu.HBM),
        out_shape=jax.ShapeDtypeStruct((_R, _C), jnp.float32),
        scratch_shapes=[
            pltpu.VMEM((_R, _C), jnp.float32),
            pltpu.VMEM((_NS, _R, _CW), jnp.float32),
            pltpu.VMEM((2, _R, _CW), jnp.float32),
            pltpu.SemaphoreType.DMA((_NM,)),
            pltpu.SemaphoreType.DMA((_NS,)),
            pltpu.SemaphoreType.DMA((2,)),
        ],
    )(output_scores.T, mask.T)
    return out_t.T


# speedup vs baseline: 1.0904x; 1.0904x over previous
import jax
import jax.numpy as jnp
from jax.experimental import pallas as pl
from jax.experimental.pallas import tpu as pltpu

_R, _C = 200, 16384
_MCW = 2048
_NM = _C // _MCW
_CW = 4096
_NC = _C // _CW
_NS = 4


def _k(s_hbm, m_hbm, o_hbm, m_v, s_b, o_b, sem_m, sem_s, sem_o):
    for q in range(_NM):
        sl = pl.ds(q * _MCW, _MCW)
        pltpu.make_async_copy(m_hbm.at[:, sl], m_v.at[:, sl], sem_m.at[q]).start()
    for c in range(_NS):
        sl = pl.ds(c * _CW, _CW)
        pltpu.make_async_copy(s_hbm.at[:, sl], s_b.at[c], sem_s.at[c]).start()

    cnt = 0.0
    for q in range(_NM):
        sl = pl.ds(q * _MCW, _MCW)
        pltpu.make_async_copy(m_hbm.at[:, sl], m_v.at[:, sl], sem_m.at[q]).wait()
        cnt = cnt + jnp.sum((m_v[:, sl] > 0).astype(jnp.float32))
    scale = 0.6931471805599453 / cnt

    for c in range(_NC):
        cur = c % _NS
        sl = pl.ds(c * _CW, _CW)
        pltpu.make_async_copy(s_hbm.at[:, sl], s_b.at[cur], sem_s.at[cur]).wait()
        if c >= 2:
            psl = pl.ds((c - 2) * _CW, _CW)
            pltpu.make_async_copy(o_b.at[c % 2], o_hbm.at[:, psl], sem_o.at[c % 2]).wait()
        t = jnp.exp2(s_b[cur] * (-1.4426950408889634))
        o_b[c % 2] = (jnp.log2(1.0 + t) * m_v[:, sl]) * scale
        pltpu.make_async_copy(o_b.at[c % 2], o_hbm.at[:, sl], sem_o.at[c % 2]).start()
        if c + _NS < _NC:
            nsl = pl.ds((c + _NS) * _CW, _CW)
            pltpu.make_async_copy(s_hbm.at[:, nsl], s_b.at[cur], sem_s.at[cur]).start()

    for c in (_NC - 2, _NC - 1):
        sl = pl.ds(c * _CW, _CW)
        pltpu.make_async_copy(o_b.at[c % 2], o_hbm.at[:, sl], sem_o.at[c % 2]).wait()


def kernel(output_scores, mask):
    out_t = pl.pallas_call(
        _k,
        in_specs=[
            pl.BlockSpec(memory_space=pltpu.HBM),
            pl.BlockSpec(memory_space=pltpu.HBM),
        ],
        out_specs=pl.BlockSpec(memory_space=pltpu.HBM),
        out_shape=jax.ShapeDtypeStruct((_R, _C), jnp.float32),
        scratch_shapes=[
            pltpu.VMEM((_R, _C), jnp.float32),
            pltpu.VMEM((_NS, _R, _CW), jnp.float32),
            pltpu.VMEM((2, _R, _CW), jnp.float32),
            pltpu.SemaphoreType.DMA((_NM,)),
            pltpu.SemaphoreType.DMA((_NS,)),
            pltpu.SemaphoreType.DMA((2,)),
        ],
    )(output_scores.T, mask.T)
    return out_t.T
